# baseline (speedup 1.0000x reference)
import dataclasses

import jax
import jax.numpy as jnp
from jax.experimental import pallas as pl
from jax.experimental.pallas import tpu as pltpu
from jax.experimental.pallas import tpu_sc as plsc


_R = 128
_W = 128
_L = 16

_C_EXPMASK = jnp.int32(0x7F800000)
_C_ONE = jnp.int32(0x3F800000)
_C_KEEP = jnp.int32(-2139095041)
_C_HALF = jnp.int32(0x3F000000)


def _sc_body(in_vmem, out_vmem):
    lane_mask = (jax.lax.iota(jnp.int32, _L) & 7) == 0

    @pl.loop(0, _R)
    def _copy(r):
        @pl.loop(0, _W, step=_L)
        def _(c):
            out_vmem[r, pl.ds(c, _L)] = in_vmem[r, pl.ds(c, _L)]

    @pl.loop(0, _R, step=8)
    def _mute(r):
        @pl.loop(0, _W, step=_L)
        def _(c):
            v = in_vmem[r, pl.ds(c, _L)]
            bits = plsc.bitcast(v, jnp.int32)
            expf = bits & _C_EXPMASK
            muted_bits = (bits & _C_KEEP) | _C_HALF
            muted = plsc.bitcast(muted_bits, jnp.float32)
            apply = lane_mask & (expf >= _C_ONE)
            out_vmem[r, pl.ds(c, _L)] = jnp.where(apply, muted, v)


def kernel(inputs):
    n, h, w = inputs.shape
    x2 = inputs.reshape(n * h, w)
    mesh = plsc.VectorSubcoreMesh(core_axis_name="c", subcore_axis_name="s")

    cp = pltpu.CompilerParams()
    if "needs_layout_passes" in pltpu.CompilerParams.__dataclass_fields__:
        cp = dataclasses.replace(cp, needs_layout_passes=False)

    @pl.kernel(out_type=jax.ShapeDtypeStruct((n * h, w), inputs.dtype),
               mesh=mesh, compiler_params=cp)
    def run(x_hbm, o_hbm):
        pltpu.emit_pipeline(
            _sc_body,
            grid=(n * h // _R,),
            in_specs=[pl.BlockSpec((_R, w), lambda i: (i, 0))],
            out_specs=[pl.BlockSpec((_R, w), lambda i: (i, 0))],
            core_axis_name=("c", "s"),
            dimension_semantics=(pltpu.PARALLEL,),
        )(x_hbm, o_hbm)

    return run(x2).reshape(n, h, w)


# speedup vs baseline: 4.0441x; 4.0441x over previous
import jax
import jax.numpy as jnp
from jax.experimental import pallas as pl


_B = 96


def _mute_block_kernel(m_ref, x_ref, o_ref):
    x = x_ref[...]
    m = m_ref[...]
    bits = jax.lax.bitcast_convert_type(x, jnp.uint32)
    absbits = bits & jnp.uint32(0x7FFFFFFF)
    muted_bits = (bits & jnp.uint32(0x807FFFFF)) | jnp.uint32(126 << 23)
    muted = jax.lax.bitcast_convert_type(muted_bits, jnp.float32)
    apply = m & (absbits >= jnp.uint32(0x3F800000))
    o_ref[...] = jnp.where(apply, muted, x)


from jax.experimental.pallas import tpu as pltpu


def kernel(inputs):
    n, h, w = inputs.shape
    mask = ((jnp.arange(h, dtype=jnp.int32) % 8 == 0)[:, None]
            & (jnp.arange(w, dtype=jnp.int32) % 8 == 0)[None, :])
    mask = mask.reshape(1, h, w)
    grid = (pl.cdiv(n, _B),)
    return pl.pallas_call(
        _mute_block_kernel,
        grid=grid,
        in_specs=[
            pl.BlockSpec((1, h, w), lambda i: (0, 0, 0)),
            pl.BlockSpec((_B, h, w), lambda i: (i, 0, 0)),
        ],
        out_specs=pl.BlockSpec((_B, h, w), lambda i: (i, 0, 0)),
        out_shape=jax.ShapeDtypeStruct(inputs.shape, inputs.dtype),
        compiler_params=pltpu.CompilerParams(
            dimension_semantics=(pltpu.PARALLEL,),
        ),
    )(mask, inputs)


# speedup vs baseline: 4.0983x; 1.0134x over previous
import jax
import jax.numpy as jnp
from jax.experimental import pallas as pl


_B = 96


def _mute_block_kernel(x_ref, o_ref):
    x = x_ref[...]
    bits = jax.lax.bitcast_convert_type(x, jnp.uint32)
    absbits = bits & jnp.uint32(0x7FFFFFFF)
    muted_bits = (bits & jnp.uint32(0x807FFFFF)) | jnp.uint32(126 << 23)
    muted = jax.lax.bitcast_convert_type(muted_bits, jnp.float32)
    shape = x.shape
    r = jax.lax.broadcasted_iota(jnp.int32, shape, 1)
    c = jax.lax.broadcasted_iota(jnp.int32, shape, 2)
    on_grid = ((r & 7) == 0) & ((c & 7) == 0)
    apply = on_grid & (absbits >= jnp.uint32(0x3F800000))
    o_ref[...] = jnp.where(apply, muted, x)


from jax.experimental.pallas import tpu as pltpu


def kernel(inputs):
    n, h, w = inputs.shape
    grid = (pl.cdiv(n, _B),)
    return pl.pallas_call(
        _mute_block_kernel,
        grid=grid,
        in_specs=[pl.BlockSpec((_B, h, w), lambda i: (i, 0, 0))],
        out_specs=pl.BlockSpec((_B, h, w), lambda i: (i, 0, 0)),
        out_shape=jax.ShapeDtypeStruct(inputs.shape, inputs.dtype),
        compiler_params=pltpu.CompilerParams(
            dimension_semantics=(pltpu.PARALLEL,),
        ),
    )(inputs)
